# baseline (speedup 1.0000x reference)
import functools

import jax
import jax.numpy as jnp
from jax import lax
from jax.experimental import pallas as pl
from jax.experimental.pallas import tpu as pltpu
from jax.experimental.pallas import tpu_sc as plsc

N = 10000
D = 128
LAM = 1.0
GAMMA = 0.0
ALP = 1.0 / (LAM + 1.0)
PROP_STEPS = 8

NPAD = 10240
NW = 32
CHUNK = 80
RPS = NPAD // 16

_mesh = plsc.VectorSubcoreMesh(core_axis_name="c", subcore_axis_name="s")


def _make_deg_kernel(cpw):
    @functools.partial(
        pl.kernel,
        mesh=_mesh,
        out_type=jax.ShapeDtypeStruct((2, NPAD, D), jnp.float32),
        scratch_types=[
            pltpu.VMEM((cpw, CHUNK), jnp.int32),
            pltpu.VMEM((CHUNK, D), jnp.float32),
            pltpu.VMEM_SHARED((NPAD, D), jnp.float32),
            pltpu.SemaphoreType.DMA,
        ],
    )
    def deg_kernel(dstp, ones_hbm, zeros_d, dacc_out, dst_v, ones_v, dacc_sh,
                   sem_s):
        c = lax.axis_index("c")
        s = lax.axis_index("s")
        w = s * 2 + c
        pltpu.sync_copy(zeros_d.at[pl.ds(s * RPS, RPS)],
                        dacc_sh.at[pl.ds(s * RPS, RPS)])
        pltpu.sync_copy(dstp.at[w], dst_v)
        pltpu.sync_copy(ones_hbm, ones_v)
        plsc.subcore_barrier()

        def body(j, carry):
            pltpu.async_copy(ones_v, dacc_sh.at[dst_v.at[j]], sem_s, add=True)
            return carry

        lax.fori_loop(0, cpw, body, 0)

        def drain(j, carry):
            pltpu.make_async_copy(ones_v, dacc_sh.at[dst_v.at[j]],
                                  sem_s).wait()
            return carry

        lax.fori_loop(0, cpw, drain, 0)
        plsc.subcore_barrier()
        pltpu.sync_copy(dacc_sh.at[pl.ds(s * RPS, RPS)],
                        dacc_out.at[c, pl.ds(s * RPS, RPS)])

    return deg_kernel


def _make_edge_kernel(cpw):
    @functools.partial(
        pl.kernel,
        mesh=_mesh,
        out_type=jax.ShapeDtypeStruct((2, NPAD, D), jnp.float32),
        scratch_types=[
            pltpu.VMEM((cpw // 2, CHUNK), jnp.int32),
            pltpu.VMEM((cpw // 2, CHUNK), jnp.int32),
            pltpu.VMEM((3, CHUNK, D), jnp.float32),
            pltpu.VMEM_SHARED((NPAD, D), jnp.float32),
            pltpu.SemaphoreType.DMA,
        ],
    )
    def edge_kernel(z, srcp, dstp, zeros_d, acc_out, src_v, dst_v, rows_v,
                    acc_sh, sem_g):
        c = lax.axis_index("c")
        s = lax.axis_index("s")
        w = s * 2 + c
        ch = cpw // 2
        pltpu.sync_copy(zeros_d.at[pl.ds(s * RPS, RPS)],
                        acc_sh.at[pl.ds(s * RPS, RPS)])
        plsc.subcore_barrier()

        for p in range(2):
            pltpu.sync_copy(srcp.at[w, pl.ds(p * ch, ch)], src_v)
            pltpu.sync_copy(dstp.at[w, pl.ds(p * ch, ch)], dst_v)

            pltpu.async_copy(z.at[src_v.at[0]], rows_v.at[0], sem_g)

            @pl.when(1 < ch)
            def _():
                pltpu.async_copy(z.at[src_v.at[1]], rows_v.at[1], sem_g)

            def body(j, carry):
                buf = lax.rem(j, 3)
                nb = lax.rem(j + 2, 3)

                @pl.when(j + 2 < ch)
                def _():
                    pltpu.async_copy(z.at[src_v.at[j + 2]], rows_v.at[nb],
                                     sem_g)

                pltpu.make_async_copy(z.at[src_v.at[j]], rows_v.at[buf],
                                      sem_g).wait()

                pltpu.sync_copy(rows_v.at[buf], acc_sh.at[dst_v.at[j]],
                                add=True)
                return carry

            lax.fori_loop(0, ch, body, 0)
        plsc.subcore_barrier()
        pltpu.sync_copy(acc_sh.at[pl.ds(s * RPS, RPS)],
                        acc_out.at[c, pl.ds(s * RPS, RPS)])

    return edge_kernel


_ROWS_BLK = 2048


NB = NPAD // _ROWS_BLK


def _prep_body(d0_ref, d1_ref, x_ref, z_ref, cz_ref, b_ref, post_ref):
    d = d0_ref[:, 0:1] + d1_ref[:, 0:1]
    dt = LAM * d + (1.0 + GAMMA)
    dts = lax.rsqrt(dt)
    dinv = 1.0 / dt
    x = x_ref[...]
    z_ref[...] = x * dts
    cz_ref[...] = (ALP * dinv * dts) * x
    shp = (x.shape[0], 8)
    b_ref[...] = jnp.broadcast_to((ALP * LAM) * dinv, shp)
    post_ref[...] = jnp.broadcast_to(jnp.sqrt(dt), shp)


def _update_body(a0_ref, a1_ref, z_ref, cz_ref, b_ref, post_ref, out_ref,
                 *, last):
    zn = ((1.0 - ALP) * z_ref[...]
          + b_ref[:, 0:1] * (a0_ref[...] + a1_ref[...]) + cz_ref[...])
    if last:
        zn = zn * post_ref[:, 0:1]
    out_ref[...] = zn


def _prep_call(dacc, xp):
    grid = (NB,)
    blk = pl.BlockSpec((_ROWS_BLK, D), lambda i: (i, 0))
    nblk = pl.BlockSpec((_ROWS_BLK, 8), lambda i: (i, 0))
    dflat = dacc.reshape(2 * NPAD, D)
    return pl.pallas_call(
        _prep_body,
        grid=grid,
        in_specs=[
            pl.BlockSpec((_ROWS_BLK, D), lambda i: (i, 0)),
            pl.BlockSpec((_ROWS_BLK, D), lambda i: (i + NB, 0)),
            blk,
        ],
        out_specs=[blk, blk, nblk, nblk],
        out_shape=[
            jax.ShapeDtypeStruct((NPAD, D), jnp.float32),
            jax.ShapeDtypeStruct((NPAD, D), jnp.float32),
            jax.ShapeDtypeStruct((NPAD, 8), jnp.float32),
            jax.ShapeDtypeStruct((NPAD, 8), jnp.float32),
        ],
    )(dflat, dflat, xp)


def _update_call(accs, z, cz, b, post, last):
    grid = (NB,)
    blk = pl.BlockSpec((_ROWS_BLK, D), lambda i: (i, 0))
    nblk = pl.BlockSpec((_ROWS_BLK, 8), lambda i: (i, 0))
    aflat = accs.reshape(2 * NPAD, D)
    return pl.pallas_call(
        functools.partial(_update_body, last=last),
        grid=grid,
        in_specs=[
            pl.BlockSpec((_ROWS_BLK, D), lambda i: (i, 0)),
            pl.BlockSpec((_ROWS_BLK, D), lambda i: (i + NB, 0)),
            blk, blk, nblk, nblk,
        ],
        out_specs=blk,
        out_shape=jax.ShapeDtypeStruct((NPAD, D), jnp.float32),
    )(aflat, aflat, z, cz, b, post)


def kernel(X, edge_index):
    e = edge_index.shape[1]
    cpw = ((-(-e // (NW * CHUNK)) + 15) // 16) * 16
    epw = cpw * CHUNK
    epad = epw * NW
    npadding = epad - e

    src = edge_index[0].astype(jnp.int32)
    dst = edge_index[1].astype(jnp.int32)
    pidx = jnp.arange(npadding, dtype=jnp.int32)
    pad_src = (pidx * 131) % N
    pad_dst = N + pidx % (NPAD - N)
    srcp = jnp.concatenate([src, pad_src]).reshape(NW, cpw, CHUNK)
    dstp = jnp.concatenate([dst, pad_dst]).reshape(NW, cpw, CHUNK)

    xp = jnp.pad(X, ((0, NPAD - N), (0, 0)))
    zeros_d = jnp.zeros((NPAD, D), jnp.float32)
    ones_d = jnp.ones((CHUNK, D), jnp.float32)

    dacc = _make_deg_kernel(cpw)(dstp, ones_d, zeros_d)
    z, cz, b, post = _prep_call(dacc, xp)
    edge_fn = _make_edge_kernel(cpw)
    for step in range(PROP_STEPS):
        accs = edge_fn(z, srcp, dstp, zeros_d)
        z = _update_call(accs, z, cz, b, post, last=(step == PROP_STEPS - 1))
    return z[:N]


# speedup vs baseline: 1.0103x; 1.0103x over previous
import functools

import jax
import jax.numpy as jnp
from jax import lax
from jax.experimental import pallas as pl
from jax.experimental.pallas import tpu as pltpu
from jax.experimental.pallas import tpu_sc as plsc

N = 10000
D = 128
LAM = 1.0
GAMMA = 0.0
ALP = 1.0 / (LAM + 1.0)
PROP_STEPS = 8

NPAD = 10240
NW = 32
CHUNK = 80
RPS = NPAD // 16

_mesh = plsc.VectorSubcoreMesh(core_axis_name="c", subcore_axis_name="s")


def _make_deg_kernel(cpw):
    @functools.partial(
        pl.kernel,
        mesh=_mesh,
        out_type=jax.ShapeDtypeStruct((2, NPAD, D), jnp.float32),
        scratch_types=[
            pltpu.VMEM((cpw, CHUNK), jnp.int32),
            pltpu.VMEM((CHUNK, D), jnp.float32),
            pltpu.VMEM_SHARED((NPAD, D), jnp.float32),
            pltpu.SemaphoreType.DMA,
        ],
    )
    def deg_kernel(dstp, ones_hbm, zeros_d, dacc_out, dst_v, ones_v, dacc_sh,
                   sem_s):
        c = lax.axis_index("c")
        s = lax.axis_index("s")
        w = s * 2 + c
        pltpu.sync_copy(zeros_d.at[pl.ds(s * RPS, RPS)],
                        dacc_sh.at[pl.ds(s * RPS, RPS)])
        pltpu.sync_copy(dstp.at[w], dst_v)
        pltpu.sync_copy(ones_hbm, ones_v)
        plsc.subcore_barrier()

        def body(j, carry):
            pltpu.async_copy(ones_v, dacc_sh.at[dst_v.at[j]], sem_s, add=True)
            return carry

        lax.fori_loop(0, cpw, body, 0)

        def drain(j, carry):
            pltpu.make_async_copy(ones_v, dacc_sh.at[dst_v.at[j]],
                                  sem_s).wait()
            return carry

        lax.fori_loop(0, cpw, drain, 0)
        plsc.subcore_barrier()
        pltpu.sync_copy(dacc_sh.at[pl.ds(s * RPS, RPS)],
                        dacc_out.at[c, pl.ds(s * RPS, RPS)])

    return deg_kernel


def _make_edge_kernel(cpw):
    @functools.partial(
        pl.kernel,
        mesh=_mesh,
        out_type=jax.ShapeDtypeStruct((2, NPAD, D), jnp.float32),
        scratch_types=[
            pltpu.VMEM((cpw // 2, CHUNK), jnp.int32),
            pltpu.VMEM((cpw // 2, CHUNK), jnp.int32),
            pltpu.VMEM((3, CHUNK, D), jnp.float32),
            pltpu.VMEM_SHARED((NPAD, D), jnp.float32),
            pltpu.SemaphoreType.DMA,
        ],
    )
    def edge_kernel(z, srcp, dstp, zeros_d, acc_out, src_v, dst_v, rows_v,
                    acc_sh, sem_g):
        c = lax.axis_index("c")
        s = lax.axis_index("s")
        w = s * 2 + c
        ch = cpw // 2
        for p in range(2):
            pltpu.sync_copy(srcp.at[w, pl.ds(p * ch, ch)], src_v)
            pltpu.sync_copy(dstp.at[w, pl.ds(p * ch, ch)], dst_v)

            pltpu.async_copy(z.at[src_v.at[0]], rows_v.at[0], sem_g)

            @pl.when(1 < ch)
            def _():
                pltpu.async_copy(z.at[src_v.at[1]], rows_v.at[1], sem_g)

            if p == 0:
                pltpu.sync_copy(zeros_d.at[pl.ds(s * RPS, RPS)],
                                acc_sh.at[pl.ds(s * RPS, RPS)])
                plsc.subcore_barrier()

            def body(j, carry):
                buf = lax.rem(j, 3)
                nb = lax.rem(j + 2, 3)

                @pl.when(j + 2 < ch)
                def _():
                    pltpu.async_copy(z.at[src_v.at[j + 2]], rows_v.at[nb],
                                     sem_g)

                pltpu.make_async_copy(z.at[src_v.at[j]], rows_v.at[buf],
                                      sem_g).wait()

                pltpu.sync_copy(rows_v.at[buf], acc_sh.at[dst_v.at[j]],
                                add=True)
                return carry

            lax.fori_loop(0, ch, body, 0)
        plsc.subcore_barrier()
        pltpu.sync_copy(acc_sh.at[pl.ds(s * RPS, RPS)],
                        acc_out.at[c, pl.ds(s * RPS, RPS)])

    return edge_kernel


_ROWS_BLK = 2048


NB = NPAD // _ROWS_BLK


def _prep_body(d0_ref, d1_ref, x_ref, z_ref, cz_ref, b_ref, post_ref):
    d = d0_ref[:, 0:1] + d1_ref[:, 0:1]
    dt = LAM * d + (1.0 + GAMMA)
    dts = lax.rsqrt(dt)
    dinv = 1.0 / dt
    x = x_ref[...]
    z_ref[...] = x * dts
    cz_ref[...] = (ALP * dinv * dts) * x
    shp = (x.shape[0], 8)
    b_ref[...] = jnp.broadcast_to((ALP * LAM) * dinv, shp)
    post_ref[...] = jnp.broadcast_to(jnp.sqrt(dt), shp)


def _update_body(a0_ref, a1_ref, z_ref, cz_ref, b_ref, post_ref, out_ref,
                 *, last):
    zn = ((1.0 - ALP) * z_ref[...]
          + b_ref[:, 0:1] * (a0_ref[...] + a1_ref[...]) + cz_ref[...])
    if last:
        zn = zn * post_ref[:, 0:1]
    out_ref[...] = zn


def _prep_call(dacc, xp):
    grid = (NB,)
    blk = pl.BlockSpec((_ROWS_BLK, D), lambda i: (i, 0))
    nblk = pl.BlockSpec((_ROWS_BLK, 8), lambda i: (i, 0))
    dflat = dacc.reshape(2 * NPAD, D)
    return pl.pallas_call(
        _prep_body,
        grid=grid,
        in_specs=[
            pl.BlockSpec((_ROWS_BLK, D), lambda i: (i, 0)),
            pl.BlockSpec((_ROWS_BLK, D), lambda i: (i + NB, 0)),
            blk,
        ],
        out_specs=[blk, blk, nblk, nblk],
        out_shape=[
            jax.ShapeDtypeStruct((NPAD, D), jnp.float32),
            jax.ShapeDtypeStruct((NPAD, D), jnp.float32),
            jax.ShapeDtypeStruct((NPAD, 8), jnp.float32),
            jax.ShapeDtypeStruct((NPAD, 8), jnp.float32),
        ],
    )(dflat, dflat, xp)


def _update_call(accs, z, cz, b, post, last):
    grid = (NB,)
    blk = pl.BlockSpec((_ROWS_BLK, D), lambda i: (i, 0))
    nblk = pl.BlockSpec((_ROWS_BLK, 8), lambda i: (i, 0))
    aflat = accs.reshape(2 * NPAD, D)
    return pl.pallas_call(
        functools.partial(_update_body, last=last),
        grid=grid,
        in_specs=[
            pl.BlockSpec((_ROWS_BLK, D), lambda i: (i, 0)),
            pl.BlockSpec((_ROWS_BLK, D), lambda i: (i + NB, 0)),
            blk, blk, nblk, nblk,
        ],
        out_specs=blk,
        out_shape=jax.ShapeDtypeStruct((NPAD, D), jnp.float32),
    )(aflat, aflat, z, cz, b, post)


def kernel(X, edge_index):
    e = edge_index.shape[1]
    cpw = ((-(-e // (NW * CHUNK)) + 15) // 16) * 16
    epw = cpw * CHUNK
    epad = epw * NW
    npadding = epad - e

    src = edge_index[0].astype(jnp.int32)
    dst = edge_index[1].astype(jnp.int32)
    pidx = jnp.arange(npadding, dtype=jnp.int32)
    pad_src = (pidx * 131) % N
    pad_dst = N + pidx % (NPAD - N)
    srcp = jnp.concatenate([src, pad_src]).reshape(NW, cpw, CHUNK)
    dstp = jnp.concatenate([dst, pad_dst]).reshape(NW, cpw, CHUNK)

    xp = jnp.pad(X, ((0, NPAD - N), (0, 0)))
    zeros_d = jnp.zeros((NPAD, D), jnp.float32)
    ones_d = jnp.ones((CHUNK, D), jnp.float32)

    dacc = _make_deg_kernel(cpw)(dstp, ones_d, zeros_d)
    z, cz, b, post = _prep_call(dacc, xp)
    edge_fn = _make_edge_kernel(cpw)
    for step in range(PROP_STEPS):
        accs = edge_fn(z, srcp, dstp, zeros_d)
        z = _update_call(accs, z, cz, b, post, last=(step == PROP_STEPS - 1))
    return z[:N]
